# baseline (speedup 1.0000x reference)
import jax
import jax.numpy as jnp
from jax.experimental import pallas as pl
from jax.experimental.pallas import tpu as pltpu

_B = 2048
_D = 128
_NCLS = 256
_DA = 512
_MARGIN = 1.0
_BLK = 1024
_P = 65536.0
_TH = 32768.0


def _dist(lhs, rhs):
    return jax.lax.dot_general(lhs, rhs, (((1,), (1,)), ((), ())),
                               preferred_element_type=jnp.float32)


def _triplet_kernel(e_ref, lab_ref, sum_ref, cnt_ref,
                    lhs_ref, rhs_ref, cmin_ref, fb_ref):
    i = pl.program_id(0)
    inf = jnp.float32(jnp.inf)

    @pl.when(i == 0)
    def _():
        e = e_ref[...]
        lab = lab_ref[...]
        sq = jnp.sum(e * e, axis=1, keepdims=True)
        sqhi = sq.astype(jnp.bfloat16)
        sqlo = (sq - sqhi.astype(jnp.float32)).astype(jnp.bfloat16)
        one = jnp.ones((_B, 1), jnp.bfloat16)
        cls = jax.lax.broadcasted_iota(jnp.int32, (1, _NCLS), 1)
        oh = jnp.where(lab == cls, 1.0, 0.0)
        zp = jnp.zeros((_B, _DA - _D - 4 - _NCLS), jnp.bfloat16)
        lhs_ref[:, 0:_D] = e.astype(jnp.bfloat16)
        lhs_ref[:, _D:_D + 1] = sqhi
        lhs_ref[:, _D + 1:_D + 2] = sqlo
        lhs_ref[:, _D + 2:_D + 3] = one
        lhs_ref[:, _D + 3:_D + 4] = one
        lhs_ref[:, _D + 4:_D + 4 + _NCLS] = oh.astype(jnp.bfloat16)
        lhs_ref[:, _D + 4 + _NCLS:] = zp
        rhs_ref[:, 0:_D] = (e * jnp.float32(-2.0)).astype(jnp.bfloat16)
        rhs_ref[:, _D:_D + 1] = one
        rhs_ref[:, _D + 1:_D + 2] = one
        rhs_ref[:, _D + 2:_D + 3] = sqhi
        rhs_ref[:, _D + 3:_D + 4] = sqlo
        rhs_ref[:, _D + 4:_D + 4 + _NCLS] = (oh * _P).astype(jnp.bfloat16)
        rhs_ref[:, _D + 4 + _NCLS:] = zp
        ncls = jnp.sum(oh, axis=0, keepdims=True)
        s1 = jnp.sum(ncls * ncls, keepdims=True)
        cnt_ref[...] = (0.5 * (s1 - jnp.float32(_B))).astype(jnp.int32)

        hm = _dist(lhs_ref[0:_BLK, :], rhs_ref[...])
        neg = jnp.min(hm, axis=1, keepdims=True)
        cmin_ref[...] = jnp.min(hm[:, _BLK:], axis=0, keepdims=True)
        fb_ref[...] = hm[0:1, _BLK:]
        neg = jnp.where(neg < _TH, neg, hm[:, 0:1] - _P)
        negt = neg + jnp.float32(_P - _MARGIN)
        x = jnp.maximum(hm - negt, 0.0)
        colv = jax.lax.broadcasted_iota(jnp.int32, (1, _BLK), 1)
        rowv = jax.lax.broadcasted_iota(jnp.int32, (_BLK, 1), 0)
        sl = jnp.sum(jnp.where(colv > rowv, x[:, 0:_BLK], 0.0),
                     keepdims=True)
        su = jnp.sum(x[:, _BLK:], keepdims=True)
        sum_ref[...] = sl + su

    @pl.when(i == 1)
    def _():
        hm = _dist(lhs_ref[_BLK:, :], rhs_ref[_BLK:, :])
        neg = jnp.minimum(jnp.min(hm, axis=1, keepdims=True),
                          cmin_ref[...].T)
        neg = jnp.where(neg < _TH, neg, fb_ref[...].T - _P)
        negt = neg + jnp.float32(_P - _MARGIN)
        x = jnp.maximum(hm - negt, 0.0)
        colv = jax.lax.broadcasted_iota(jnp.int32, (1, _BLK), 1)
        rowv = jax.lax.broadcasted_iota(jnp.int32, (_BLK, 1), 0)
        total = sum_ref[...] + jnp.sum(jnp.where(colv > rowv, x, 0.0),
                                       keepdims=True)
        sum_ref[...] = total / cnt_ref[...].astype(jnp.float32)


def kernel(embeddings, target):
    labc = target.reshape(_B, 1)
    out_sum, out_cnt = pl.pallas_call(
        _triplet_kernel,
        grid=(2,),
        in_specs=[
            pl.BlockSpec((_B, _D), lambda i: (0, 0)),
            pl.BlockSpec((_B, 1), lambda i: (0, 0)),
        ],
        out_specs=[
            pl.BlockSpec((1, 1), lambda i: (0, 0)),
            pl.BlockSpec((1, 1), lambda i: (0, 0)),
        ],
        out_shape=[
            jax.ShapeDtypeStruct((1, 1), jnp.float32),
            jax.ShapeDtypeStruct((1, 1), jnp.int32),
        ],
        scratch_shapes=[
            pltpu.VMEM((_B, _DA), jnp.bfloat16),
            pltpu.VMEM((_B, _DA), jnp.bfloat16),
            pltpu.VMEM((1, _BLK), jnp.float32),
            pltpu.VMEM((1, _BLK), jnp.float32),
        ],
    )(embeddings, labc)
    return (out_sum[0, 0], out_cnt[0, 0])


# speedup vs baseline: 2.1120x; 2.1120x over previous
import jax
import jax.numpy as jnp
from jax.experimental import pallas as pl
from jax.experimental.pallas import tpu as pltpu

_B = 2048
_D = 128
_DA = _D + 8
_NCLS = 256
_MARGIN = 1.0
_H = 1024


def _dist(lhs, rhs):
    return jax.lax.dot_general(lhs, rhs, (((1,), (1,)), ((), ())),
                               preferred_element_type=jnp.float32)


def _triplet_kernel(e_ref, labc_ref, labr_ref, sum_ref, cnt_ref,
                    lhs_ref, rhs_ref):
    labr = labr_ref[...]
    labc = labc_ref[...]
    inf = jnp.float32(jnp.inf)

    e = e_ref[...]
    sq = jnp.sum(e * e, axis=1, keepdims=True)
    one = jnp.ones((_B, 1), jnp.float32)
    zp = jnp.zeros((_B, _DA - _D - 2), jnp.float32)
    lhs_ref[:, 0:_D] = e
    lhs_ref[:, _D:_D + 1] = sq
    lhs_ref[:, _D + 1:_D + 2] = one
    lhs_ref[:, _D + 2:] = zp
    rhs_ref[:, 0:_D] = e * jnp.float32(-2.0)
    rhs_ref[:, _D:_D + 1] = one
    rhs_ref[:, _D + 1:_D + 2] = sq
    rhs_ref[:, _D + 2:] = zp
    cls = jax.lax.broadcasted_iota(jnp.int32, (_NCLS, 1), 0)
    ohc = jnp.where(cls == labr, 1.0, 0.0)
    ncls = jnp.sum(ohc, axis=1, keepdims=True)
    s1 = jnp.sum(ncls * ncls, keepdims=True)
    cnt = (0.5 * (s1 - jnp.float32(_B))).astype(jnp.int32)
    cnt_ref[...] = cnt

    colv = jax.lax.broadcasted_iota(jnp.int32, (1, _H), 1)
    rowv = jax.lax.broadcasted_iota(jnp.int32, (_H, 1), 0)

    dista = _dist(lhs_ref[0:_H, :], rhs_ref[...])
    eqa = labc[0:_H] == labr
    mha = jnp.where(eqa, inf, dista)
    nega = jnp.min(mha, axis=1, keepdims=True)
    nega = jnp.where(nega < inf, nega, dista[:, 0:1])
    xa = jnp.maximum(dista - (nega - _MARGIN), 0.0)
    posl = jnp.logical_and(eqa[:, 0:_H], colv > rowv)
    sl = jnp.sum(jnp.where(posl, xa[:, 0:_H], 0.0), keepdims=True)
    su = jnp.sum(jnp.where(eqa[:, _H:], xa[:, _H:], 0.0), keepdims=True)

    distb = _dist(lhs_ref[_H:, :], rhs_ref[_H:, :])
    eqb = labc[_H:] == labr[:, _H:]
    mhb = jnp.where(eqb, inf, distb)
    cmin = jnp.min(mha[:, _H:], axis=0, keepdims=True)
    negb = jnp.minimum(jnp.min(mhb, axis=1, keepdims=True), cmin.T)
    negb = jnp.where(negb < inf, negb, dista[0:1, _H:].T)
    xb = jnp.maximum(distb - (negb - _MARGIN), 0.0)
    posb = jnp.logical_and(eqb, colv > rowv)
    sb = jnp.sum(jnp.where(posb, xb, 0.0), keepdims=True)

    sum_ref[...] = (sl + su + sb) / cnt.astype(jnp.float32)


def kernel(embeddings, target):
    labc = target.reshape(_B, 1)
    labr = target.reshape(1, _B)
    out_sum, out_cnt = pl.pallas_call(
        _triplet_kernel,
        out_shape=[
            jax.ShapeDtypeStruct((1, 1), jnp.float32),
            jax.ShapeDtypeStruct((1, 1), jnp.int32),
        ],
        scratch_shapes=[
            pltpu.VMEM((_B, _DA), jnp.float32),
            pltpu.VMEM((_B, _DA), jnp.float32),
        ],
    )(embeddings, labc, labr)
    return (out_sum[0, 0], out_cnt[0, 0])


# speedup vs baseline: 2.1638x; 1.0245x over previous
import jax
import jax.numpy as jnp
from jax.experimental import pallas as pl
from jax.experimental.pallas import tpu as pltpu

_B = 2048
_D = 128
_DA = _D + 8
_NCLS = 256
_MARGIN = 1.0
_NQ = 4
_S = _B // _NQ


def _dist(lhs, rhs):
    return jax.lax.dot_general(lhs, rhs, (((1,), (1,)), ((), ())),
                               preferred_element_type=jnp.float32)


def _triplet_kernel(e_ref, labc_ref, labr_ref, sum_ref, cnt_ref,
                    lhs_ref, rhs_ref):
    labr = labr_ref[...]
    labc = labc_ref[...]
    inf = jnp.float32(jnp.inf)

    e = e_ref[...]
    sq = jnp.sum(e * e, axis=1, keepdims=True)
    one = jnp.ones((_B, 1), jnp.float32)
    zp = jnp.zeros((_B, _DA - _D - 2), jnp.float32)
    lhs_ref[:, 0:_D] = e
    lhs_ref[:, _D:_D + 1] = sq
    lhs_ref[:, _D + 1:_D + 2] = one
    lhs_ref[:, _D + 2:] = zp
    rhs_ref[:, 0:_D] = e * jnp.float32(-2.0)
    rhs_ref[:, _D:_D + 1] = one
    rhs_ref[:, _D + 1:_D + 2] = sq
    rhs_ref[:, _D + 2:] = zp
    cls = jax.lax.broadcasted_iota(jnp.int32, (_NCLS, 1), 0)
    ohc = jnp.where(cls == labr, 1.0, 0.0)
    ncls = jnp.sum(ohc, axis=1, keepdims=True)
    s1 = jnp.sum(ncls * ncls, keepdims=True)
    cnt = (0.5 * (s1 - jnp.float32(_B))).astype(jnp.int32)
    cnt_ref[...] = cnt

    colv = jax.lax.broadcasted_iota(jnp.int32, (1, _S), 1)
    rowv = jax.lax.broadcasted_iota(jnp.int32, (_S, 1), 0)

    sums = []
    cmins = {q: [] for q in range(_NQ)}
    fbrow = None
    for q in range(_NQ):
        c0 = q * _S
        dist_q = _dist(lhs_ref[c0:c0 + _S, :], rhs_ref[c0:, :])
        eq_q = labc[c0:c0 + _S] == labr[:, c0:]
        mh_q = jnp.where(eq_q, inf, dist_q)
        if q == 0:
            fbrow = dist_q[0:1, :]
        neg = jnp.min(mh_q, axis=1, keepdims=True)
        for t in range(q + 1, _NQ):
            off = t * _S - c0
            cmins[t].append(
                jnp.min(mh_q[:, off:off + _S], axis=0, keepdims=True).T)
        for cm in cmins[q]:
            neg = jnp.minimum(neg, cm)
        fbv = dist_q[:, 0:1] if q == 0 else fbrow[0:1, c0:c0 + _S].T
        neg = jnp.where(neg < inf, neg, fbv)
        x = jnp.maximum(dist_q - (neg - _MARGIN), 0.0)
        posd = jnp.logical_and(eq_q[:, 0:_S], colv > rowv)
        sums.append(jnp.sum(jnp.where(posd, x[:, 0:_S], 0.0),
                            keepdims=True))
        if q + 1 < _NQ:
            sums.append(jnp.sum(jnp.where(eq_q[:, _S:], x[:, _S:], 0.0),
                                keepdims=True))

    total = sums[0]
    for s in sums[1:]:
        total = total + s
    sum_ref[...] = total / cnt.astype(jnp.float32)


def kernel(embeddings, target):
    labc = target.reshape(_B, 1)
    labr = target.reshape(1, _B)
    out_sum, out_cnt = pl.pallas_call(
        _triplet_kernel,
        out_shape=[
            jax.ShapeDtypeStruct((1, 1), jnp.float32),
            jax.ShapeDtypeStruct((1, 1), jnp.int32),
        ],
        scratch_shapes=[
            pltpu.VMEM((_B, _DA), jnp.float32),
            pltpu.VMEM((_B, _DA), jnp.float32),
        ],
    )(embeddings, labc, labr)
    return (out_sum[0, 0], out_cnt[0, 0])
